# baseline (speedup 1.0000x reference)
import math

import jax
import jax.numpy as jnp
from jax.experimental import pallas as pl
from jax.experimental.pallas import tpu as pltpu


def _ffn_kernel(x_ref, c_ref, o_ref, *, H, W):
    N = H * W
    C = x_ref.shape[-1]
    hid = c_ref.shape[-1]
    K = 9 * C

    x3 = x_ref[0].astype(jnp.bfloat16).reshape(H, W, C)
    zc = jnp.zeros((H, 1, C), x3.dtype)
    xm = jnp.concatenate([zc, x3[:, :W - 1]], axis=1)
    xp = jnp.concatenate([x3[:, 1:], zc], axis=1)
    x9 = jnp.concatenate([xm, x3, xp], axis=2)

    zr = jnp.zeros((1, W, 3 * C), x3.dtype)
    xdn = jnp.concatenate([zr, x9[:H - 1]], axis=0)
    xup = jnp.concatenate([x9[1:], zr], axis=0)
    x27 = jnp.concatenate([xdn, x9, xup], axis=2).reshape(N, 9 * C)

    t = jnp.dot(x27, c_ref[:K], preferred_element_type=jnp.float32)
    t = t + c_ref[K:K + N].astype(jnp.float32)

    g = t * (1.0 + jax.lax.erf(t))

    out = jnp.dot(g.astype(jnp.bfloat16), c_ref[K + N:K + N + hid],
                  preferred_element_type=jnp.float32)
    b2row = c_ref[K + N + hid:K + N + hid + 1, :C].astype(jnp.float32)
    o_ref[0] = out[:, :C] + b2row


def _fused_ffn(x, consts, *, H, W, interpret=False):
    B, N, C = x.shape
    hid = consts.shape[-1]
    assert N == H * W

    def body(*refs):
        _ffn_kernel(*refs, H=H, W=W)

    return pl.pallas_call(
        body,
        out_shape=jax.ShapeDtypeStruct((B, N, C), jnp.float32),
        grid_spec=pltpu.PrefetchScalarGridSpec(
            num_scalar_prefetch=0,
            grid=(B,),
            in_specs=[
                pl.BlockSpec((1, N, C), lambda b: (b, 0, 0)),
                pl.BlockSpec(consts.shape, lambda b: (0, 0)),
            ],
            out_specs=pl.BlockSpec((1, N, C), lambda b: (b, 0, 0)),
        ),
        compiler_params=pltpu.CompilerParams(
            dimension_semantics=("parallel",),
            vmem_limit_bytes=100 * 1024 * 1024,
        ),
        cost_estimate=pl.CostEstimate(
            flops=2 * B * N * 9 * C * hid + 2 * B * N * hid * hid,
            transcendentals=B * N * hid,
            bytes_accessed=(B * N * C * 4 + B * N * C * 4
                            + consts.shape[0] * hid * 2),
        ),
        interpret=interpret,
    )(x, consts)


def _prep_weights(w1, b1, w2, b2, dw_w, dw_b, H, W):
    C, hid = w1.shape
    inv_sqrt2 = 0.7071067811865476

    w27 = (w1[None, None] * dw_w[:, :, None, :]) * inv_sqrt2
    w27 = w27.reshape(9 * C, hid).astype(jnp.bfloat16)

    ksum = dw_w.sum((0, 1))
    row0, row2 = dw_w[0].sum(0), dw_w[2].sum(0)
    col0, col2 = dw_w[:, 0].sum(0), dw_w[:, 2].sum(0)
    eh = jnp.zeros((H, 1, 1), jnp.float32)
    top = eh.at[0].set(1.0)
    bot = eh.at[H - 1].set(1.0)
    ew = jnp.zeros((1, W, 1), jnp.float32)
    lef = ew.at[:, 0].set(1.0)
    rig = ew.at[:, W - 1].set(1.0)
    miss = (top * row0 + bot * row2 + lef * col0 + rig * col2
            - top * lef * dw_w[0, 0] - top * rig * dw_w[0, 2]
            - bot * lef * dw_w[2, 0] - bot * rig * dw_w[2, 2])
    bias_field = (dw_b + b1 * (ksum - miss)) * inv_sqrt2
    bias_field = bias_field.reshape(H * W, hid)

    w2s = w2 * (0.5 * math.sqrt(2.0))
    w2s = jnp.concatenate(
        [w2s, jnp.zeros((hid, hid - C), jnp.float32)], axis=1)
    b2row = jnp.concatenate([b2, jnp.zeros((hid - C,), jnp.float32)]
                            ).reshape(1, hid)
    b2row = jnp.concatenate([b2row, jnp.zeros((7, hid), jnp.float32)], axis=0)

    consts = jnp.concatenate([w27.astype(jnp.float32), bias_field, w2s, b2row],
                             axis=0).astype(jnp.bfloat16)
    return consts


def kernel(x, w1, b1, w2, b2, dw_w, dw_b):
    B, N, C = x.shape
    H = W = math.isqrt(N)
    consts = _prep_weights(w1, b1, w2, b2, dw_w, dw_b, H, W)
    return _fused_ffn(x, consts, H=H, W=W)
